# Initial kernel scaffold; baseline (speedup 1.0000x reference)
import jax
import jax.numpy as jnp
from jax.experimental import pallas as pl


def kernel(x, edge_index, edge_attr, u, batch, W1, b1, g1, be1, W2, b2, W3, b3, g2, be2, W4, b4):
    raise NotImplementedError("write your pallas kernel here")


# speedup vs baseline: 2.3059x; 2.3059x over previous
import functools
import jax
import jax.numpy as jnp
from jax import lax
from jax.experimental import pallas as pl
from jax.experimental.pallas import tpu as pltpu
from jax.experimental.pallas import tpu_sc as plsc

N = 10000
E = 320000
DF = 128
DE = 16
H = 128
G = 64
NG = 64
EPS = 1e-5

NC = 2
NS = 16
NW = NC * NS
EW = E // NW
B = 80
NB = EW // B
RT = N // NS

_MESH = dict(core_axis_name="c", subcore_axis_name="s", num_cores=NC,
             num_subcores=NS)
_SC_PARAMS = pltpu.CompilerParams(use_tc_tiling_on_sc=False)


def _worker(cid, sid):
  return sid * NC + cid


def _sc_stats_body(row_h, col_h, ea_h, z16_h, o16_h,
                   a_out, cr_out, cc_out,
                   a_sh, cr_sh, cc_sh, ridx_v, cidx_v, ea_v, ones_v):
  cid = lax.axis_index("c")
  sid = lax.axis_index("s")
  wid = _worker(cid, sid)
  rows0 = sid * RT

  pltpu.sync_copy(z16_h.at[pl.ds(rows0, RT)], a_sh.at[pl.ds(rows0, RT)])
  pltpu.sync_copy(z16_h.at[pl.ds(rows0, RT)], cr_sh.at[pl.ds(rows0, RT)])
  pltpu.sync_copy(z16_h.at[pl.ds(rows0, RT)], cc_sh.at[pl.ds(rows0, RT)])
  pltpu.sync_copy(o16_h, ones_v)
  plsc.subcore_barrier()

  def step(g, carry):
    base = wid * EW + g * B
    pltpu.sync_copy(row_h.at[pl.ds(base, B)], ridx_v)
    pltpu.sync_copy(col_h.at[pl.ds(base, B)], cidx_v)
    pltpu.sync_copy(ea_h.at[pl.ds(base, B)], ea_v)
    pltpu.sync_copy(ea_v, a_sh.at[ridx_v], add=True)
    pltpu.sync_copy(ones_v, cr_sh.at[ridx_v], add=True)
    pltpu.sync_copy(ones_v, cc_sh.at[cidx_v], add=True)
    return carry

  lax.fori_loop(0, NB, step, 0)
  plsc.subcore_barrier()

  pltpu.sync_copy(a_sh.at[pl.ds(rows0, RT)], a_out.at[cid, pl.ds(rows0, RT)])
  pltpu.sync_copy(cr_sh.at[pl.ds(rows0, RT)], cr_out.at[cid, pl.ds(rows0, RT)])
  pltpu.sync_copy(cc_sh.at[pl.ds(rows0, RT)], cc_out.at[cid, pl.ds(rows0, RT)])


_sc_stats = pl.kernel(
    _sc_stats_body,
    out_type=[
        jax.ShapeDtypeStruct((NC, N, 16), jnp.float32),
        jax.ShapeDtypeStruct((NC, N, 16), jnp.float32),
        jax.ShapeDtypeStruct((NC, N, 16), jnp.float32),
    ],
    mesh=plsc.VectorSubcoreMesh(**_MESH),
    compiler_params=_SC_PARAMS,
    scratch_types=[
        pltpu.VMEM_SHARED((N, 16), jnp.float32),
        pltpu.VMEM_SHARED((N, 16), jnp.float32),
        pltpu.VMEM_SHARED((N, 16), jnp.float32),
        pltpu.VMEM((B,), jnp.int32),
        pltpu.VMEM((B,), jnp.int32),
        pltpu.VMEM((B, 16), jnp.float32),
        pltpu.VMEM((B, 16), jnp.float32),
    ],
)


def _sc_main_body(row_h, col_h, ep_h, y_h, scale_h, shift_h, z128_h,
                  s_out,
                  s_sh, ridx_v, cidx_v, yv, epv, sc_v, sh_v, sem):
  cid = lax.axis_index("c")
  sid = lax.axis_index("s")
  wid = _worker(cid, sid)
  rows0 = sid * RT

  pltpu.sync_copy(z128_h.at[pl.ds(rows0, RT)], s_sh.at[pl.ds(rows0, RT)])
  pltpu.sync_copy(scale_h, sc_v)
  pltpu.sync_copy(shift_h, sh_v)
  plsc.subcore_barrier()

  scale_r = [sc_v[pl.ds(16 * j, 16)] for j in range(H // 16)]
  shift_r = [sh_v[pl.ds(16 * j, 16)] for j in range(H // 16)]

  def step(g, carry):
    base = wid * EW + g * B
    pltpu.sync_copy(row_h.at[pl.ds(base, B)], ridx_v)
    pltpu.sync_copy(col_h.at[pl.ds(base, B)], cidx_v)
    pltpu.async_copy(y_h.at[ridx_v], yv, sem).wait()
    pltpu.sync_copy(ep_h.at[pl.ds(base, B)], epv)

    def erow(e, c2):
      for j in range(H // 16):
        sl = pl.ds(16 * j, 16)
        h = yv[e, sl] + epv[e, sl]
        yv[e, sl] = jnp.maximum(h * scale_r[j] + shift_r[j], 0.0)
      return c2

    lax.fori_loop(0, B, erow, 0)
    pltpu.sync_copy(yv, s_sh.at[cidx_v], add=True)
    return carry

  lax.fori_loop(0, NB, step, 0)
  plsc.subcore_barrier()
  pltpu.sync_copy(s_sh.at[pl.ds(rows0, RT)], s_out.at[cid, pl.ds(rows0, RT)])


_sc_main = pl.kernel(
    _sc_main_body,
    out_type=[jax.ShapeDtypeStruct((NC, N, H), jnp.float32)],
    mesh=plsc.VectorSubcoreMesh(**_MESH),
    compiler_params=_SC_PARAMS,
    scratch_types=[
        pltpu.VMEM_SHARED((N, H), jnp.float32),
        pltpu.VMEM((B,), jnp.int32),
        pltpu.VMEM((B,), jnp.int32),
        pltpu.VMEM((B, H), jnp.float32),
        pltpu.VMEM((B, H), jnp.float32),
        pltpu.VMEM((H,), jnp.float32),
        pltpu.VMEM((H,), jnp.float32),
        pltpu.SemaphoreType.DMA,
    ],
)


def _y_body(x_ref, w_ref, b_ref, y_ref):
  y_ref[...] = jnp.dot(x_ref[...], w_ref[...],
                       preferred_element_type=jnp.float32) + b_ref[...]


def _tc_y(x, W1aT, b1):
  return pl.pallas_call(
      _y_body,
      out_shape=jax.ShapeDtypeStruct((N, H), jnp.float32),
  )(x, W1aT, b1)


_EB = 2000


def _ep_body(ea_ref, w_ref, ep_ref, m_ref):
  ea = ea_ref[...]
  ep_ref[...] = jnp.dot(ea, w_ref[...], preferred_element_type=jnp.float32)
  part = lax.dot_general(ea, ea, (((0,), (0,)), ((), ())),
                         preferred_element_type=jnp.float32)

  @pl.when(pl.program_id(0) == 0)
  def _():
    m_ref[...] = jnp.zeros_like(m_ref)

  m_ref[...] += part


def _tc_ep(ea, W1bT):
  return pl.pallas_call(
      _ep_body,
      grid=(E // _EB,),
      in_specs=[
          pl.BlockSpec((_EB, DE), lambda i: (i, 0)),
          pl.BlockSpec((DE, H), lambda i: (0, 0)),
      ],
      out_specs=[
          pl.BlockSpec((_EB, H), lambda i: (i, 0)),
          pl.BlockSpec((DE, DE), lambda i: (0, 0)),
      ],
      out_shape=[
          jax.ShapeDtypeStruct((E, H), jnp.float32),
          jax.ShapeDtypeStruct((DE, DE), jnp.float32),
      ],
  )(ea, W1bT)


def _stats_body(y_ref, a_ref, cr_ref, m_ref, wbt_ref, g1_ref, be1_ref, ss_ref):
  y = y_ref[...]
  c = cr_ref[0, :, 0:1] + cr_ref[1, :, 0:1]
  A = a_ref[0] + a_ref[1]
  wbt = wbt_ref[...]
  Ap = jnp.dot(A, wbt, preferred_element_type=jnp.float32)
  S1 = jnp.sum(y * c, axis=0, keepdims=True)
  Q1 = jnp.sum(y * y * c, axis=0, keepdims=True)
  X2 = jnp.sum(y * Ap, axis=0, keepdims=True)
  sAp = jnp.sum(Ap, axis=0, keepdims=True)
  B2 = jnp.dot(m_ref[...], wbt, preferred_element_type=jnp.float32)
  T2 = jnp.sum(wbt * B2, axis=0, keepdims=True)
  mu = (S1 + sAp) * (1.0 / E)
  var = (Q1 + 2.0 * X2 + T2) * (1.0 / E) - mu * mu
  scale = g1_ref[...] / jnp.sqrt(var + EPS)
  shift = be1_ref[...] - mu * scale
  ss_ref[0:1, :] = scale
  ss_ref[1:2, :] = shift


def _tc_stats(y, Ap_, CRp, M16, W1bT, g1, be1):
  return pl.pallas_call(
      _stats_body,
      out_shape=jax.ShapeDtypeStruct((2, H), jnp.float32),
  )(y, Ap_, CRp, M16, W1bT, g1, be1)


def _final_body(x_ref, s_ref, cc_ref, u_ref, batch_ref,
                w2t_ref, b2_ref, w3at_ref, w3bt_ref, w3ct_ref, b3_ref,
                g2_ref, be2_ref, w4t_ref, b4_ref, out_ref):
  sums = s_ref[0] + s_ref[1]
  cnt = cc_ref[0, :, 0:1] + cc_ref[1, :, 0:1]
  inv = 1.0 / jnp.maximum(cnt, 1.0)
  mask = (cnt > 0.0).astype(jnp.float32)
  agg = (jnp.dot(sums, w2t_ref[...], preferred_element_type=jnp.float32)
         * inv + b2_ref[...] * mask)
  bvec = batch_ref[...]
  oh = (bvec == lax.broadcasted_iota(jnp.int32, (N, NG), 1)
        ).astype(jnp.float32)
  P = jnp.dot(u_ref[...], w3ct_ref[...], preferred_element_type=jnp.float32)
  z = (jnp.dot(x_ref[...], w3at_ref[...], preferred_element_type=jnp.float32)
       + jnp.dot(agg, w3bt_ref[...], preferred_element_type=jnp.float32)
       + jnp.dot(oh, P, preferred_element_type=jnp.float32)
       + b3_ref[...])
  mu2 = jnp.mean(z, axis=0, keepdims=True)
  zc = z - mu2
  var2 = jnp.mean(zc * zc, axis=0, keepdims=True)
  zn = jnp.maximum(zc / jnp.sqrt(var2 + EPS) * g2_ref[...] + be2_ref[...],
                   0.0)
  out_ref[...] = jnp.dot(zn, w4t_ref[...],
                         preferred_element_type=jnp.float32) + b4_ref[...]


def _tc_final(x, Sp, CCp, u, batch2, W2T, b2, W3aT, W3bT, W3cT, b3, g2, be2,
              W4T, b4):
  return pl.pallas_call(
      _final_body,
      out_shape=jax.ShapeDtypeStruct((N, DF), jnp.float32),
  )(x, Sp, CCp, u, batch2, W2T, b2, W3aT, W3bT, W3cT, b3, g2, be2, W4T, b4)


def kernel(x, edge_index, edge_attr, u, batch,
           W1, b1, g1, be1, W2, b2,
           W3, b3, g2, be2, W4, b4):
  row = edge_index[0]
  col = edge_index[1]
  W1aT = W1[:, :DF].T
  W1bT = W1[:, DF:].T
  b1r = b1.reshape(1, H)

  z16 = jnp.zeros((N, 16), jnp.float32)
  o16 = jnp.ones((B, 16), jnp.float32)
  z128 = jnp.zeros((N, H), jnp.float32)

  Ap_, CRp, CCp = _sc_stats(row, col, edge_attr, z16, o16)
  y = _tc_y(x, W1aT, b1r)
  ep, M16 = _tc_ep(edge_attr, W1bT)
  ss = _tc_stats(y, Ap_, CRp, M16, W1bT, g1.reshape(1, H), be1.reshape(1, H))
  scale = ss[0]
  shift = ss[1]
  (Sp,) = _sc_main(row, col, ep, y, scale, shift, z128)
  out = _tc_final(x, Sp, CCp, u, batch.reshape(N, 1),
                  W2.T, b2.reshape(1, H), W3[:, :DF].T, W3[:, DF:DF + H].T,
                  W3[:, DF + H:].T, b3.reshape(1, H), g2.reshape(1, H),
                  be2.reshape(1, H), W4.T, b4.reshape(1, DF))
  return out


# speedup vs baseline: 3.7252x; 1.6155x over previous
import functools
import jax
import jax.numpy as jnp
from jax import lax
from jax.experimental import pallas as pl
from jax.experimental.pallas import tpu as pltpu
from jax.experimental.pallas import tpu_sc as plsc

N = 10000
E = 320000
DF = 128
DE = 16
H = 128
G = 64
NG = 64
EPS = 1e-5

NC = 2
NS = 16
NW = NC * NS
EW = E // NW
B = 40
NB = EW // B
SL = 5
SL2 = 2
CH = 50
B1 = 80
NB1 = EW // B1
RT = N // NS

_MESH = dict(core_axis_name="c", subcore_axis_name="s", num_cores=NC,
             num_subcores=NS)
_SC_PARAMS = pltpu.CompilerParams(use_tc_tiling_on_sc=False)


def _worker(cid, sid):
  return sid * NC + cid


def _sc_stats_body(row2_h, col2_h, ea_h, z16_h, o16_h,
                   a_out, cr_out, cc_out,
                   a_sh, cr_sh, cc_sh, rbuf, cbuf, eav, ones_v,
                   esem, asem, rsem, csem):
  cid = lax.axis_index("c")
  sid = lax.axis_index("s")
  wid = _worker(cid, sid)
  rows0 = sid * RT

  pltpu.sync_copy(z16_h.at[pl.ds(rows0, RT)], a_sh.at[pl.ds(rows0, RT)])
  pltpu.sync_copy(z16_h.at[pl.ds(rows0, RT)], cr_sh.at[pl.ds(rows0, RT)])
  pltpu.sync_copy(z16_h.at[pl.ds(rows0, RT)], cc_sh.at[pl.ds(rows0, RT)])
  pltpu.sync_copy(o16_h, ones_v)
  pltpu.sync_copy(row2_h.at[pl.ds(wid * NB1, NB1)], rbuf)
  pltpu.sync_copy(col2_h.at[pl.ds(wid * NB1, NB1)], cbuf)
  plsc.subcore_barrier()

  def ea_load(g, s):
    base = wid * EW + g * B1
    return pltpu.make_async_copy(ea_h.at[pl.ds(base, B1)], eav.at[s],
                                 esem.at[s])

  for s in range(SL):
    ea_load(s, s).start()

  def step(k, carry):
    for s in range(SL):
      g = k * SL + s
      ea_load(g, s).wait()

      @pl.when(g >= SL)
      def _():
        pltpu.make_async_copy(eav.at[s], a_sh.at[rbuf.at[g]], asem.at[s]).wait()
        pltpu.make_async_copy(ones_v, cr_sh.at[rbuf.at[g]], rsem.at[s]).wait()
        pltpu.make_async_copy(ones_v, cc_sh.at[cbuf.at[g]], csem.at[s]).wait()

      pltpu.async_copy(eav.at[s], a_sh.at[rbuf.at[g]], asem.at[s], add=True)
      pltpu.async_copy(ones_v, cr_sh.at[rbuf.at[g]], rsem.at[s], add=True)
      pltpu.async_copy(ones_v, cc_sh.at[cbuf.at[g]], csem.at[s], add=True)

      @pl.when(g + SL < NB1)
      def _():
        ea_load(g + SL, s).start()
    return carry

  lax.fori_loop(0, NB1 // SL, step, 0)
  for s in range(SL):
    g = NB1 - SL + s
    pltpu.make_async_copy(eav.at[s], a_sh.at[rbuf.at[g]], asem.at[s]).wait()
    pltpu.make_async_copy(ones_v, cr_sh.at[rbuf.at[g]], rsem.at[s]).wait()
    pltpu.make_async_copy(ones_v, cc_sh.at[cbuf.at[g]], csem.at[s]).wait()
  plsc.subcore_barrier()

  pltpu.sync_copy(a_sh.at[pl.ds(rows0, RT)], a_out.at[cid, pl.ds(rows0, RT)])
  pltpu.sync_copy(cr_sh.at[pl.ds(rows0, RT)], cr_out.at[cid, pl.ds(rows0, RT)])
  pltpu.sync_copy(cc_sh.at[pl.ds(rows0, RT)], cc_out.at[cid, pl.ds(rows0, RT)])


_sc_stats = pl.kernel(
    _sc_stats_body,
    out_type=[
        jax.ShapeDtypeStruct((NC, N, 16), jnp.float32),
        jax.ShapeDtypeStruct((NC, N, 16), jnp.float32),
        jax.ShapeDtypeStruct((NC, N, 16), jnp.float32),
    ],
    mesh=plsc.VectorSubcoreMesh(**_MESH),
    compiler_params=_SC_PARAMS,
    scratch_types=[
        pltpu.VMEM_SHARED((N, 16), jnp.float32),
        pltpu.VMEM_SHARED((N, 16), jnp.float32),
        pltpu.VMEM_SHARED((N, 16), jnp.float32),
        pltpu.VMEM((NB1, B1), jnp.int32),
        pltpu.VMEM((NB1, B1), jnp.int32),
        pltpu.VMEM((SL, B1, 16), jnp.float32),
        pltpu.VMEM((B1, 16), jnp.float32),
        pltpu.SemaphoreType.DMA((SL,)),
        pltpu.SemaphoreType.DMA((SL,)),
        pltpu.SemaphoreType.DMA((SL,)),
        pltpu.SemaphoreType.DMA((SL,)),
    ],
)


def _sc_main_body(row2_h, col2_h, ep_h, y_h, scale_h, shift_h, z128_h,
                  s_out,
                  s_sh, rbuf, cbuf, yv, epv, ov, sc_v, sh_v,
                  gsem, esem, ssem):
  cid = lax.axis_index("c")
  sid = lax.axis_index("s")
  wid = _worker(cid, sid)
  rows0 = sid * RT

  pltpu.sync_copy(z128_h.at[pl.ds(rows0, RT)], s_sh.at[pl.ds(rows0, RT)])
  pltpu.sync_copy(scale_h, sc_v)
  pltpu.sync_copy(shift_h, sh_v)
  plsc.subcore_barrier()

  scale_r = [sc_v[pl.ds(16 * j, 16)] for j in range(H // 16)]
  shift_r = [sh_v[pl.ds(16 * j, 16)] for j in range(H // 16)]

  def y_gather(g, s):
    return pltpu.make_async_copy(y_h.at[rbuf.at[g]], yv.at[s], gsem.at[s])

  def ep_load(c, g, s):
    base = wid * EW + (c * CH + g) * B
    return pltpu.make_async_copy(ep_h.at[pl.ds(base, B)], epv.at[s],
                                 esem.at[s])

  def scatter(g, s):
    return pltpu.make_async_copy(ov.at[s], s_sh.at[cbuf.at[g]], ssem.at[s])

  def chunk(c, carry):
    pltpu.sync_copy(row2_h.at[pl.ds(wid * NB + c * CH, CH)], rbuf)
    pltpu.sync_copy(col2_h.at[pl.ds(wid * NB + c * CH, CH)], cbuf)
    for s in range(SL2):
      y_gather(s, s).start()
      ep_load(c, s, s).start()

    def step(k, c2):
      for s in range(SL2):
        g = k * SL2 + s
        y_gather(g, s).wait()
        ep_load(c, g, s).wait()

        @pl.when(g >= SL2)
        def _():
          scatter(g, s).wait()

        def erow(e, c3):
          for j in range(H // 16):
            sl = pl.ds(16 * j, 16)
            h = yv[s, e, sl] + epv[s, e, sl]
            ov[s, e, sl] = jnp.maximum(h * scale_r[j] + shift_r[j], 0.0)
          return c3

        lax.fori_loop(0, B, erow, 0)

        @pl.when(g + SL2 < CH)
        def _():
          y_gather(g + SL2, s).start()
          ep_load(c, g + SL2, s).start()

        pltpu.async_copy(ov.at[s], s_sh.at[cbuf.at[g]], ssem.at[s], add=True)
      return c2

    lax.fori_loop(0, CH // SL2, step, 0)
    for s in range(SL2):
      scatter(CH - SL2 + s, s).wait()
    return carry

  lax.fori_loop(0, NB // CH, chunk, 0)
  plsc.subcore_barrier()
  pltpu.sync_copy(s_sh.at[pl.ds(rows0, RT)], s_out.at[cid, pl.ds(rows0, RT)])


_sc_main = pl.kernel(
    _sc_main_body,
    out_type=[jax.ShapeDtypeStruct((NC, N, H), jnp.float32)],
    mesh=plsc.VectorSubcoreMesh(**_MESH),
    compiler_params=_SC_PARAMS,
    scratch_types=[
        pltpu.VMEM_SHARED((N, H), jnp.float32),
        pltpu.VMEM((CH, B), jnp.int32),
        pltpu.VMEM((CH, B), jnp.int32),
        pltpu.VMEM((SL2, B, H), jnp.float32),
        pltpu.VMEM((SL2, B, H), jnp.float32),
        pltpu.VMEM((SL2, B, H), jnp.float32),
        pltpu.VMEM((H,), jnp.float32),
        pltpu.VMEM((H,), jnp.float32),
        pltpu.SemaphoreType.DMA((SL2,)),
        pltpu.SemaphoreType.DMA((SL2,)),
        pltpu.SemaphoreType.DMA((SL2,)),
    ],
)


def _y_body(x_ref, w_ref, b_ref, y_ref):
  y_ref[...] = jnp.dot(x_ref[...], w_ref[...],
                       preferred_element_type=jnp.float32) + b_ref[...]


def _tc_y(x, W1aT, b1):
  return pl.pallas_call(
      _y_body,
      out_shape=jax.ShapeDtypeStruct((N, H), jnp.float32),
  )(x, W1aT, b1)


_EB = 2000


def _ep_body(ea_ref, w_ref, ep_ref, m_ref):
  ea = ea_ref[...]
  ep_ref[...] = jnp.dot(ea, w_ref[...], preferred_element_type=jnp.float32)
  part = lax.dot_general(ea, ea, (((0,), (0,)), ((), ())),
                         preferred_element_type=jnp.float32)

  @pl.when(pl.program_id(0) == 0)
  def _():
    m_ref[...] = jnp.zeros_like(m_ref)

  m_ref[...] += part


def _tc_ep(ea, W1bT):
  return pl.pallas_call(
      _ep_body,
      grid=(E // _EB,),
      in_specs=[
          pl.BlockSpec((_EB, DE), lambda i: (i, 0)),
          pl.BlockSpec((DE, H), lambda i: (0, 0)),
      ],
      out_specs=[
          pl.BlockSpec((_EB, H), lambda i: (i, 0)),
          pl.BlockSpec((DE, DE), lambda i: (0, 0)),
      ],
      out_shape=[
          jax.ShapeDtypeStruct((E, H), jnp.float32),
          jax.ShapeDtypeStruct((DE, DE), jnp.float32),
      ],
  )(ea, W1bT)


def _stats_body(y_ref, a_ref, cr_ref, m_ref, wbt_ref, g1_ref, be1_ref, ss_ref):
  y = y_ref[...]
  c = cr_ref[0, :, 0:1] + cr_ref[1, :, 0:1]
  A = a_ref[0] + a_ref[1]
  wbt = wbt_ref[...]
  Ap = jnp.dot(A, wbt, preferred_element_type=jnp.float32)
  S1 = jnp.sum(y * c, axis=0, keepdims=True)
  Q1 = jnp.sum(y * y * c, axis=0, keepdims=True)
  X2 = jnp.sum(y * Ap, axis=0, keepdims=True)
  sAp = jnp.sum(Ap, axis=0, keepdims=True)
  B2 = jnp.dot(m_ref[...], wbt, preferred_element_type=jnp.float32)
  T2 = jnp.sum(wbt * B2, axis=0, keepdims=True)
  mu = (S1 + sAp) * (1.0 / E)
  var = (Q1 + 2.0 * X2 + T2) * (1.0 / E) - mu * mu
  scale = g1_ref[...] / jnp.sqrt(var + EPS)
  shift = be1_ref[...] - mu * scale
  ss_ref[0:1, :] = scale
  ss_ref[1:2, :] = shift


def _tc_stats(y, Ap_, CRp, M16, W1bT, g1, be1):
  return pl.pallas_call(
      _stats_body,
      out_shape=jax.ShapeDtypeStruct((2, H), jnp.float32),
  )(y, Ap_, CRp, M16, W1bT, g1, be1)


def _final_body(x_ref, s_ref, cc_ref, u_ref, batch_ref,
                w2t_ref, b2_ref, w3at_ref, w3bt_ref, w3ct_ref, b3_ref,
                g2_ref, be2_ref, w4t_ref, b4_ref, out_ref):
  sums = s_ref[0] + s_ref[1]
  cnt = cc_ref[0, :, 0:1] + cc_ref[1, :, 0:1]
  inv = 1.0 / jnp.maximum(cnt, 1.0)
  mask = (cnt > 0.0).astype(jnp.float32)
  agg = (jnp.dot(sums, w2t_ref[...], preferred_element_type=jnp.float32)
         * inv + b2_ref[...] * mask)
  bvec = batch_ref[...]
  oh = (bvec == lax.broadcasted_iota(jnp.int32, (N, NG), 1)
        ).astype(jnp.float32)
  P = jnp.dot(u_ref[...], w3ct_ref[...], preferred_element_type=jnp.float32)
  z = (jnp.dot(x_ref[...], w3at_ref[...], preferred_element_type=jnp.float32)
       + jnp.dot(agg, w3bt_ref[...], preferred_element_type=jnp.float32)
       + jnp.dot(oh, P, preferred_element_type=jnp.float32)
       + b3_ref[...])
  mu2 = jnp.mean(z, axis=0, keepdims=True)
  zc = z - mu2
  var2 = jnp.mean(zc * zc, axis=0, keepdims=True)
  zn = jnp.maximum(zc / jnp.sqrt(var2 + EPS) * g2_ref[...] + be2_ref[...],
                   0.0)
  out_ref[...] = jnp.dot(zn, w4t_ref[...],
                         preferred_element_type=jnp.float32) + b4_ref[...]


def _tc_final(x, Sp, CCp, u, batch2, W2T, b2, W3aT, W3bT, W3cT, b3, g2, be2,
              W4T, b4):
  return pl.pallas_call(
      _final_body,
      out_shape=jax.ShapeDtypeStruct((N, DF), jnp.float32),
  )(x, Sp, CCp, u, batch2, W2T, b2, W3aT, W3bT, W3cT, b3, g2, be2, W4T, b4)


def kernel(x, edge_index, edge_attr, u, batch,
           W1, b1, g1, be1, W2, b2,
           W3, b3, g2, be2, W4, b4):
  row = edge_index[0]
  col = edge_index[1]
  row1 = row.reshape(E // B1, B1)
  col1 = col.reshape(E // B1, B1)
  row2 = row.reshape(E // B, B)
  col2 = col.reshape(E // B, B)
  W1aT = W1[:, :DF].T
  W1bT = W1[:, DF:].T
  b1r = b1.reshape(1, H)

  z16 = jnp.zeros((N, 16), jnp.float32)
  o16 = jnp.ones((B1, 16), jnp.float32)
  z128 = jnp.zeros((N, H), jnp.float32)

  Ap_, CRp, CCp = _sc_stats(row1, col1, edge_attr, z16, o16)
  y = _tc_y(x, W1aT, b1r)
  ep, M16 = _tc_ep(edge_attr, W1bT)
  ss = _tc_stats(y, Ap_, CRp, M16, W1bT, g1.reshape(1, H), be1.reshape(1, H))
  scale = ss[0]
  shift = ss[1]
  (Sp,) = _sc_main(row2, col2, ep, y, scale, shift, z128)
  out = _tc_final(x, Sp, CCp, u, batch.reshape(N, 1),
                  W2.T, b2.reshape(1, H), W3[:, :DF].T, W3[:, DF:DF + H].T,
                  W3[:, DF + H:].T, b3.reshape(1, H), g2.reshape(1, H),
                  be2.reshape(1, H), W4.T, b4.reshape(1, DF))
  return out


# speedup vs baseline: 3.7317x; 1.0017x over previous
import functools
import jax
import jax.numpy as jnp
from jax import lax
from jax.experimental import pallas as pl
from jax.experimental.pallas import tpu as pltpu
from jax.experimental.pallas import tpu_sc as plsc

N = 10000
E = 320000
DF = 128
DE = 16
H = 128
G = 64
NG = 64
EPS = 1e-5

NC = 2
NS = 16
NW = NC * NS
EW = E // NW
B = 40
NB = EW // B
SL2 = 2
CH = 50
B1 = 80
NB1 = EW // B1
RT = N // NS

_MESH = dict(core_axis_name="c", subcore_axis_name="s", num_cores=NC,
             num_subcores=NS)
_SC_PARAMS = pltpu.CompilerParams(use_tc_tiling_on_sc=False)


def _worker(cid, sid):
  return sid * NC + cid


def _sc_stats_body(row2_h, col2_h, ea_h, z16_h, o16_h,
                   a_out, cr_out, cc_out,
                   a_sh, cr_sh, cc_sh, rbuf, cbuf, eav, ones_v, esem):
  cid = lax.axis_index("c")
  sid = lax.axis_index("s")
  wid = _worker(cid, sid)
  rows0 = sid * RT

  pltpu.sync_copy(z16_h.at[pl.ds(rows0, RT)], a_sh.at[pl.ds(rows0, RT)])
  pltpu.sync_copy(z16_h.at[pl.ds(rows0, RT)], cr_sh.at[pl.ds(rows0, RT)])
  pltpu.sync_copy(z16_h.at[pl.ds(rows0, RT)], cc_sh.at[pl.ds(rows0, RT)])
  pltpu.sync_copy(o16_h, ones_v)
  pltpu.sync_copy(row2_h.at[pl.ds(wid * NB1, NB1)], rbuf)
  pltpu.sync_copy(col2_h.at[pl.ds(wid * NB1, NB1)], cbuf)
  plsc.subcore_barrier()

  def ea_load(g, j):
    base = wid * EW + g * B1
    return pltpu.make_async_copy(ea_h.at[pl.ds(base, B1)], eav.at[j],
                                 esem.at[j])

  ea_load(0, 0).start()

  def step(k, carry):
    for j in range(2):
      g = k * 2 + j
      ea_load(g, j).wait()
      ea_load(g + 1, 1 - j).start()
      pltpu.sync_copy(eav.at[j], a_sh.at[rbuf.at[g]], add=True)
      pltpu.sync_copy(ones_v, cr_sh.at[rbuf.at[g]], add=True)
      pltpu.sync_copy(ones_v, cc_sh.at[cbuf.at[g]], add=True)
    return carry

  lax.fori_loop(0, (NB1 - 1) // 2, step, 0)
  g_last = NB1 - 1
  ea_load(g_last, g_last % 2).wait()
  pltpu.sync_copy(eav.at[g_last % 2], a_sh.at[rbuf.at[g_last]], add=True)
  pltpu.sync_copy(ones_v, cr_sh.at[rbuf.at[g_last]], add=True)
  pltpu.sync_copy(ones_v, cc_sh.at[cbuf.at[g_last]], add=True)
  plsc.subcore_barrier()

  pltpu.sync_copy(a_sh.at[pl.ds(rows0, RT)], a_out.at[cid, pl.ds(rows0, RT)])
  pltpu.sync_copy(cr_sh.at[pl.ds(rows0, RT)], cr_out.at[cid, pl.ds(rows0, RT)])
  pltpu.sync_copy(cc_sh.at[pl.ds(rows0, RT)], cc_out.at[cid, pl.ds(rows0, RT)])


_sc_stats = pl.kernel(
    _sc_stats_body,
    out_type=[
        jax.ShapeDtypeStruct((NC, N, 16), jnp.float32),
        jax.ShapeDtypeStruct((NC, N, 16), jnp.float32),
        jax.ShapeDtypeStruct((NC, N, 16), jnp.float32),
    ],
    mesh=plsc.VectorSubcoreMesh(**_MESH),
    compiler_params=_SC_PARAMS,
    scratch_types=[
        pltpu.VMEM_SHARED((N, 16), jnp.float32),
        pltpu.VMEM_SHARED((N, 16), jnp.float32),
        pltpu.VMEM_SHARED((N, 16), jnp.float32),
        pltpu.VMEM((NB1, B1), jnp.int32),
        pltpu.VMEM((NB1, B1), jnp.int32),
        pltpu.VMEM((2, B1, 16), jnp.float32),
        pltpu.VMEM((B1, 16), jnp.float32),
        pltpu.SemaphoreType.DMA((2,)),
    ],
)


def _sc_main_body(row2_h, col2_h, ep_h, y_h, scale_h, shift_h, z128_h,
                  s_out,
                  s_sh, rbuf, cbuf, yv, epv, ov, sc_v, sh_v,
                  gsem, esem, ssem):
  cid = lax.axis_index("c")
  sid = lax.axis_index("s")
  wid = _worker(cid, sid)
  rows0 = sid * RT

  pltpu.sync_copy(z128_h.at[pl.ds(rows0, RT)], s_sh.at[pl.ds(rows0, RT)])
  pltpu.sync_copy(scale_h, sc_v)
  pltpu.sync_copy(shift_h, sh_v)
  plsc.subcore_barrier()

  scale_r = [sc_v[pl.ds(16 * j, 16)] for j in range(H // 16)]
  shift_r = [sh_v[pl.ds(16 * j, 16)] for j in range(H // 16)]

  def y_gather(g, s):
    return pltpu.make_async_copy(y_h.at[rbuf.at[g]], yv.at[s], gsem.at[s])

  def ep_load(c, g, s):
    base = wid * EW + (c * CH + g) * B
    return pltpu.make_async_copy(ep_h.at[pl.ds(base, B)], epv.at[s],
                                 esem.at[s])

  def scatter(g, s):
    return pltpu.make_async_copy(ov.at[s], s_sh.at[cbuf.at[g]], ssem.at[s])

  def chunk(c, carry):
    pltpu.sync_copy(row2_h.at[pl.ds(wid * NB + c * CH, CH)], rbuf)
    pltpu.sync_copy(col2_h.at[pl.ds(wid * NB + c * CH, CH)], cbuf)
    for s in range(SL2):
      y_gather(s, s).start()
      ep_load(c, s, s).start()

    def step(k, c2):
      for s in range(SL2):
        g = k * SL2 + s
        y_gather(g, s).wait()
        ep_load(c, g, s).wait()

        @pl.when(g >= SL2)
        def _():
          scatter(g, s).wait()

        def erow(e, c3):
          for j in range(H // 16):
            sl = pl.ds(16 * j, 16)
            h = yv[s, e, sl] + epv[s, e, sl]
            ov[s, e, sl] = jnp.maximum(h * scale_r[j] + shift_r[j], 0.0)
          return c3

        lax.fori_loop(0, B, erow, 0)

        @pl.when(g + SL2 < CH)
        def _():
          y_gather(g + SL2, s).start()
          ep_load(c, g + SL2, s).start()

        pltpu.async_copy(ov.at[s], s_sh.at[cbuf.at[g]], ssem.at[s], add=True)
      return c2

    lax.fori_loop(0, CH // SL2, step, 0)
    for s in range(SL2):
      scatter(CH - SL2 + s, s).wait()
    return carry

  lax.fori_loop(0, NB // CH, chunk, 0)
  plsc.subcore_barrier()
  pltpu.sync_copy(s_sh.at[pl.ds(rows0, RT)], s_out.at[cid, pl.ds(rows0, RT)])


_sc_main = pl.kernel(
    _sc_main_body,
    out_type=[jax.ShapeDtypeStruct((NC, N, H), jnp.float32)],
    mesh=plsc.VectorSubcoreMesh(**_MESH),
    compiler_params=_SC_PARAMS,
    scratch_types=[
        pltpu.VMEM_SHARED((N, H), jnp.float32),
        pltpu.VMEM((CH, B), jnp.int32),
        pltpu.VMEM((CH, B), jnp.int32),
        pltpu.VMEM((SL2, B, H), jnp.float32),
        pltpu.VMEM((SL2, B, H), jnp.float32),
        pltpu.VMEM((SL2, B, H), jnp.float32),
        pltpu.VMEM((H,), jnp.float32),
        pltpu.VMEM((H,), jnp.float32),
        pltpu.SemaphoreType.DMA((SL2,)),
        pltpu.SemaphoreType.DMA((SL2,)),
        pltpu.SemaphoreType.DMA((SL2,)),
    ],
)


def _y_body(x_ref, w_ref, b_ref, y_ref):
  y_ref[...] = jnp.dot(x_ref[...], w_ref[...],
                       preferred_element_type=jnp.float32) + b_ref[...]


def _tc_y(x, W1aT, b1):
  return pl.pallas_call(
      _y_body,
      out_shape=jax.ShapeDtypeStruct((N, H), jnp.float32),
  )(x, W1aT, b1)


_EB = 2000


def _ep_body(ea_ref, w_ref, ep_ref, m_ref):
  ea = ea_ref[...]
  ep_ref[...] = jnp.dot(ea, w_ref[...], preferred_element_type=jnp.float32)
  part = lax.dot_general(ea, ea, (((0,), (0,)), ((), ())),
                         preferred_element_type=jnp.float32)

  @pl.when(pl.program_id(0) == 0)
  def _():
    m_ref[...] = jnp.zeros_like(m_ref)

  m_ref[...] += part


def _tc_ep(ea, W1bT):
  return pl.pallas_call(
      _ep_body,
      grid=(E // _EB,),
      in_specs=[
          pl.BlockSpec((_EB, DE), lambda i: (i, 0)),
          pl.BlockSpec((DE, H), lambda i: (0, 0)),
      ],
      out_specs=[
          pl.BlockSpec((_EB, H), lambda i: (i, 0)),
          pl.BlockSpec((DE, DE), lambda i: (0, 0)),
      ],
      out_shape=[
          jax.ShapeDtypeStruct((E, H), jnp.float32),
          jax.ShapeDtypeStruct((DE, DE), jnp.float32),
      ],
  )(ea, W1bT)


def _stats_body(y_ref, a_ref, cr_ref, m_ref, wbt_ref, g1_ref, be1_ref, ss_ref):
  y = y_ref[...]
  c = cr_ref[0, :, 0:1] + cr_ref[1, :, 0:1]
  A = a_ref[0] + a_ref[1]
  wbt = wbt_ref[...]
  Ap = jnp.dot(A, wbt, preferred_element_type=jnp.float32)
  S1 = jnp.sum(y * c, axis=0, keepdims=True)
  Q1 = jnp.sum(y * y * c, axis=0, keepdims=True)
  X2 = jnp.sum(y * Ap, axis=0, keepdims=True)
  sAp = jnp.sum(Ap, axis=0, keepdims=True)
  B2 = jnp.dot(m_ref[...], wbt, preferred_element_type=jnp.float32)
  T2 = jnp.sum(wbt * B2, axis=0, keepdims=True)
  mu = (S1 + sAp) * (1.0 / E)
  var = (Q1 + 2.0 * X2 + T2) * (1.0 / E) - mu * mu
  scale = g1_ref[...] / jnp.sqrt(var + EPS)
  shift = be1_ref[...] - mu * scale
  ss_ref[0:1, :] = scale
  ss_ref[1:2, :] = shift


def _tc_stats(y, Ap_, CRp, M16, W1bT, g1, be1):
  return pl.pallas_call(
      _stats_body,
      out_shape=jax.ShapeDtypeStruct((2, H), jnp.float32),
  )(y, Ap_, CRp, M16, W1bT, g1, be1)


def _final_body(x_ref, s_ref, cc_ref, u_ref, batch_ref,
                w2t_ref, b2_ref, w3at_ref, w3bt_ref, w3ct_ref, b3_ref,
                g2_ref, be2_ref, w4t_ref, b4_ref, out_ref):
  sums = s_ref[0] + s_ref[1]
  cnt = cc_ref[0, :, 0:1] + cc_ref[1, :, 0:1]
  inv = 1.0 / jnp.maximum(cnt, 1.0)
  mask = (cnt > 0.0).astype(jnp.float32)
  agg = (jnp.dot(sums, w2t_ref[...], preferred_element_type=jnp.float32)
         * inv + b2_ref[...] * mask)
  bvec = batch_ref[...]
  oh = (bvec == lax.broadcasted_iota(jnp.int32, (N, NG), 1)
        ).astype(jnp.float32)
  P = jnp.dot(u_ref[...], w3ct_ref[...], preferred_element_type=jnp.float32)
  z = (jnp.dot(x_ref[...], w3at_ref[...], preferred_element_type=jnp.float32)
       + jnp.dot(agg, w3bt_ref[...], preferred_element_type=jnp.float32)
       + jnp.dot(oh, P, preferred_element_type=jnp.float32)
       + b3_ref[...])
  mu2 = jnp.mean(z, axis=0, keepdims=True)
  zc = z - mu2
  var2 = jnp.mean(zc * zc, axis=0, keepdims=True)
  zn = jnp.maximum(zc / jnp.sqrt(var2 + EPS) * g2_ref[...] + be2_ref[...],
                   0.0)
  out_ref[...] = jnp.dot(zn, w4t_ref[...],
                         preferred_element_type=jnp.float32) + b4_ref[...]


def _tc_final(x, Sp, CCp, u, batch2, W2T, b2, W3aT, W3bT, W3cT, b3, g2, be2,
              W4T, b4):
  return pl.pallas_call(
      _final_body,
      out_shape=jax.ShapeDtypeStruct((N, DF), jnp.float32),
  )(x, Sp, CCp, u, batch2, W2T, b2, W3aT, W3bT, W3cT, b3, g2, be2, W4T, b4)


def kernel(x, edge_index, edge_attr, u, batch,
           W1, b1, g1, be1, W2, b2,
           W3, b3, g2, be2, W4, b4):
  row = edge_index[0]
  col = edge_index[1]
  row1 = row.reshape(E // B1, B1)
  col1 = col.reshape(E // B1, B1)
  row2 = row.reshape(E // B, B)
  col2 = col.reshape(E // B, B)
  W1aT = W1[:, :DF].T
  W1bT = W1[:, DF:].T
  b1r = b1.reshape(1, H)

  z16 = jnp.zeros((N, 16), jnp.float32)
  o16 = jnp.ones((B1, 16), jnp.float32)
  z128 = jnp.zeros((N, H), jnp.float32)

  Ap_, CRp, CCp = _sc_stats(row1, col1, edge_attr, z16, o16)
  y = _tc_y(x, W1aT, b1r)
  ep, M16 = _tc_ep(edge_attr, W1bT)
  ss = _tc_stats(y, Ap_, CRp, M16, W1bT, g1.reshape(1, H), be1.reshape(1, H))
  scale = ss[0]
  shift = ss[1]
  (Sp,) = _sc_main(row2, col2, ep, y, scale, shift, z128)
  out = _tc_final(x, Sp, CCp, u, batch.reshape(N, 1),
                  W2.T, b2.reshape(1, H), W3[:, :DF].T, W3[:, DF:DF + H].T,
                  W3[:, DF + H:].T, b3.reshape(1, H), g2.reshape(1, H),
                  be2.reshape(1, H), W4.T, b4.reshape(1, DF))
  return out
